# baseline (speedup 1.0000x reference)
import functools

import jax
import jax.numpy as jnp
from jax import lax
from jax.experimental import pallas as pl
from jax.experimental.pallas import tpu as pltpu
from jax.experimental.pallas import tpu_sc as plsc

DIM = 64
NUM_WORKERS = 32
WIDEN_BLOCK = 4096
NBUF = 4


def _row_major_table(token_table):
    V, D = token_table.shape
    t_t = token_table.T
    H = WIDEN_BLOCK // 2
    grid = (V + WIDEN_BLOCK - 1) // WIDEN_BLOCK
    last_blk = (V - 1) // H
    eye = jnp.eye(D, dtype=jnp.float32)

    def wk(ta_ref, tb_ref, eye_ref, o_ref):
        def tr(blk):
            return jax.lax.dot_general(
                blk, eye_ref[...], (((0,), (0,)), ((), ())),
                preferred_element_type=jnp.float32,
                precision=jax.lax.Precision.HIGHEST)

        o_ref[...] = jnp.concatenate(
            [tr(ta_ref[...]), tr(tb_ref[...])], axis=1)

    return pl.pallas_call(
        wk,
        grid=(grid,),
        in_specs=[
            pl.BlockSpec((D, H), lambda i: (0, 2 * i)),
            pl.BlockSpec((D, H), lambda i: (0, jnp.minimum(2 * i + 1, last_blk))),
            pl.BlockSpec((D, D), lambda i: (0, 0)),
        ],
        out_specs=pl.BlockSpec((H, 2 * D), lambda i: (i, 0)),
        out_shape=jax.ShapeDtypeStruct((grid * H, 2 * D), jnp.float32),
    )(t_t, t_t, eye)


def _remap_indices(x, width):
    h = width // 2
    i = x // width
    r = x % width
    return i * width + (r % h) * 2 + r // h


def kernel(x, token_table, pos_table):
    B, S = x.shape
    V, D = token_table.shape
    assert B % NUM_WORKERS == 0 and D == DIM
    seqs_per_w = B // NUM_WORKERS
    assert seqs_per_w % NBUF == 0
    SP = 256

    x_pad = jnp.pad(_remap_indices(x.astype(jnp.int32), WIDEN_BLOCK),
                    ((0, 0), (0, SP - S)))
    packed = _row_major_table(token_table)
    table_rm = packed.reshape(2 * packed.shape[0], D)

    mesh = plsc.VectorSubcoreMesh(core_axis_name="c", subcore_axis_name="s")

    @functools.partial(
        pl.kernel,
        mesh=mesh,
        out_type=jax.ShapeDtypeStruct((B, S, 2 * D), jnp.float32),
        compiler_params=pltpu.CompilerParams(use_tc_tiling_on_sc=False),
        scratch_types=[
            pltpu.VMEM((S, DIM), jnp.float32),
            [pltpu.VMEM((SP,), jnp.int32) for _ in range(NBUF)],
            [pltpu.VMEM((S, DIM), jnp.float32) for _ in range(NBUF)],
            [pltpu.SemaphoreType.DMA for _ in range(NBUF)],
            [pltpu.SemaphoreType.DMA for _ in range(NBUF)],
            [pltpu.SemaphoreType.DMA for _ in range(NBUF)],
        ],
    )
    def k(x_hbm, wide_hbm, pos_hbm, out_hbm, pos_v, idxs, toks, gsems, osems,
          isems):
        wid = lax.axis_index("s") * 2 + lax.axis_index("c")
        base = wid * seqs_per_w
        pltpu.sync_copy(pos_hbm.at[pl.ds(0, S)], pos_v)

        def fire_idx(c, s):
            pltpu.async_copy(x_hbm.at[base + c], idxs[s], isems[s])

        def wait_idx(c, s):
            pltpu.make_async_copy(x_hbm.at[base + c], idxs[s], isems[s]).wait()

        def fire_gather(s):
            pltpu.async_copy(
                wide_hbm.at[idxs[s].at[pl.ds(0, 128)]],
                toks[s].at[pl.ds(0, 128)], gsems[s])
            pltpu.async_copy(
                wide_hbm.at[idxs[s].at[pl.ds(128, S - 128)]],
                toks[s].at[pl.ds(128, S - 128)], gsems[s])

        def wait_gather(s):
            pltpu.make_async_copy(
                wide_hbm.at[idxs[s].at[pl.ds(0, 128)]],
                toks[s].at[pl.ds(0, 128)], gsems[s]).wait()
            pltpu.make_async_copy(
                wide_hbm.at[idxs[s].at[pl.ds(128, S - 128)]],
                toks[s].at[pl.ds(128, S - 128)], gsems[s]).wait()

        def fire_scatter(c, s):
            pltpu.async_copy(
                toks[s], out_hbm.at[base + c, :, pl.ds(0, DIM)], osems[s])

        def wait_scatter(c, s):
            pltpu.make_async_copy(
                toks[s], out_hbm.at[base + c, :, pl.ds(0, DIM)],
                osems[s]).wait()

        def add_chunk(s):
            tok = toks[s]

            def add_body(j, carry):
                for t in range(DIM // 16):
                    sl = pl.ds(t * 16, 16)
                    plsc.addupdate(tok.at[j, sl], pos_v[j, sl])
                return carry

            lax.fori_loop(0, S, add_body, 0, unroll=4)

        for s in range(NBUF - 1):
            pltpu.sync_copy(x_hbm.at[base + s], idxs[s])
            fire_gather(s)
        fire_idx(NBUF - 1, NBUF - 1)

        def body(i, carry):
            for s in range(NBUF):
                c = NBUF * i + s
                wait_gather(s)

                @pl.when(c + NBUF < seqs_per_w)
                def _():
                    fire_idx(c + NBUF, s)
                add_chunk(s)
                fire_scatter(c, s)
                sp = (s + NBUF - 1) % NBUF

                @pl.when(c >= 1)
                def _():
                    wait_scatter(c - 1, sp)

                @pl.when(c + NBUF - 1 < seqs_per_w)
                def _():
                    wait_idx(c + NBUF - 1, sp)
                    fire_gather(sp)
            return carry

        lax.fori_loop(0, seqs_per_w // NBUF, body, 0)
        wait_scatter(seqs_per_w - 1, (seqs_per_w - 1) % NBUF)

    out2 = k(x_pad, table_rm, pos_table)
    return out2[:, :, :DIM]


# speedup vs baseline: 1.0796x; 1.0796x over previous
import functools

import jax
import jax.numpy as jnp
from jax import lax
from jax.experimental import pallas as pl
from jax.experimental.pallas import tpu as pltpu
from jax.experimental.pallas import tpu_sc as plsc

DIM = 64
NUM_WORKERS = 32
WIDEN_BLOCK = 4096
NBUF = 4


def _row_major_table(token_table):
    V, D = token_table.shape
    t_t = token_table.T
    grid = (V + WIDEN_BLOCK - 1) // WIDEN_BLOCK
    eye = jnp.eye(D, dtype=jnp.float32)

    def wk(t_ref, eye_ref, o_ref):
        blk = t_ref[...]
        tt = jax.lax.dot_general(
            blk, eye_ref[...], (((0,), (0,)), ((), ())),
            preferred_element_type=jnp.float32,
            precision=jax.lax.Precision.HIGHEST)
        o_ref[:, 0:D] = tt

    return pl.pallas_call(
        wk,
        grid=(grid,),
        in_specs=[
            pl.BlockSpec((D, WIDEN_BLOCK), lambda i: (0, i)),
            pl.BlockSpec((D, D), lambda i: (0, 0)),
        ],
        out_specs=pl.BlockSpec((WIDEN_BLOCK, 2 * D), lambda i: (i, 0)),
        out_shape=jax.ShapeDtypeStruct((V, 2 * D), jnp.float32),
    )(t_t, eye)


def kernel(x, token_table, pos_table):
    B, S = x.shape
    V, D = token_table.shape
    assert B % NUM_WORKERS == 0 and D == DIM
    seqs_per_w = B // NUM_WORKERS
    assert seqs_per_w % NBUF == 0
    SP = 256

    x_pad = jnp.pad(x.astype(jnp.int32) * 2, ((0, 0), (0, SP - S)))
    table_rm = _row_major_table(token_table).reshape(2 * V, D)

    mesh = plsc.VectorSubcoreMesh(core_axis_name="c", subcore_axis_name="s")

    @functools.partial(
        pl.kernel,
        mesh=mesh,
        out_type=jax.ShapeDtypeStruct((B, S, 2 * D), jnp.float32),
        compiler_params=pltpu.CompilerParams(use_tc_tiling_on_sc=False),
        scratch_types=[
            pltpu.VMEM((S, DIM), jnp.float32),
            [pltpu.VMEM((SP,), jnp.int32) for _ in range(NBUF)],
            [pltpu.VMEM((S, DIM), jnp.float32) for _ in range(NBUF)],
            [pltpu.SemaphoreType.DMA for _ in range(NBUF)],
            [pltpu.SemaphoreType.DMA for _ in range(NBUF)],
            [pltpu.SemaphoreType.DMA for _ in range(NBUF)],
        ],
    )
    def k(x_hbm, wide_hbm, pos_hbm, out_hbm, pos_v, idxs, toks, gsems, osems,
          isems):
        wid = lax.axis_index("s") * 2 + lax.axis_index("c")
        base = wid * seqs_per_w
        pltpu.sync_copy(pos_hbm.at[pl.ds(0, S)], pos_v)

        def fire_idx(c, s):
            pltpu.async_copy(x_hbm.at[base + c], idxs[s], isems[s])

        def wait_idx(c, s):
            pltpu.make_async_copy(x_hbm.at[base + c], idxs[s], isems[s]).wait()

        def fire_gather(s):
            pltpu.async_copy(
                wide_hbm.at[idxs[s].at[pl.ds(0, 128)]],
                toks[s].at[pl.ds(0, 128)], gsems[s])
            pltpu.async_copy(
                wide_hbm.at[idxs[s].at[pl.ds(128, S - 128)]],
                toks[s].at[pl.ds(128, S - 128)], gsems[s])

        def wait_gather(s):
            pltpu.make_async_copy(
                wide_hbm.at[idxs[s].at[pl.ds(0, 128)]],
                toks[s].at[pl.ds(0, 128)], gsems[s]).wait()
            pltpu.make_async_copy(
                wide_hbm.at[idxs[s].at[pl.ds(128, S - 128)]],
                toks[s].at[pl.ds(128, S - 128)], gsems[s]).wait()

        def fire_scatter(c, s):
            pltpu.async_copy(
                toks[s], out_hbm.at[base + c, :, pl.ds(0, DIM)], osems[s])

        def wait_scatter(c, s):
            pltpu.make_async_copy(
                toks[s], out_hbm.at[base + c, :, pl.ds(0, DIM)],
                osems[s]).wait()

        def add_chunk(s):
            tok = toks[s]

            def add_body(j, carry):
                for t in range(DIM // 16):
                    sl = pl.ds(t * 16, 16)
                    plsc.addupdate(tok.at[j, sl], pos_v[j, sl])
                return carry

            lax.fori_loop(0, S, add_body, 0, unroll=4)

        for s in range(NBUF - 1):
            pltpu.sync_copy(x_hbm.at[base + s], idxs[s])
            fire_gather(s)
        fire_idx(NBUF - 1, NBUF - 1)

        def body(i, carry):
            for s in range(NBUF):
                c = NBUF * i + s
                wait_gather(s)

                @pl.when(c + NBUF < seqs_per_w)
                def _():
                    fire_idx(c + NBUF, s)
                add_chunk(s)
                fire_scatter(c, s)
                sp = (s + NBUF - 1) % NBUF

                @pl.when(c >= 1)
                def _():
                    wait_scatter(c - 1, sp)

                @pl.when(c + NBUF - 1 < seqs_per_w)
                def _():
                    wait_idx(c + NBUF - 1, sp)
                    fire_gather(sp)
            return carry

        lax.fori_loop(0, seqs_per_w // NBUF, body, 0)
        wait_scatter(seqs_per_w - 1, (seqs_per_w - 1) % NBUF)

    out2 = k(x_pad, table_rm, pos_table)
    return out2[:, :, :DIM]
